# baseline (speedup 1.0000x reference)
import functools

import jax
import jax.numpy as jnp
from jax.experimental import pallas as pl
from jax.experimental.pallas import tpu as pltpu

BLK = 1024
NBUF = 12


def _decode_kernel(f_ref, w_ref, b_ref, o_ref, w_buf, sem, *, nk: int, d: int):
    steps = 2 * nk

    def copy(s):
        return pltpu.make_async_copy(
            w_ref.at[pl.ds(s * BLK, BLK), :],
            w_buf.at[s % NBUF],
            sem.at[s % NBUF],
        )

    AHEAD = NBUF - 2
    for j in range(AHEAD):
        copy(j).start()
    o_ref[...] = jnp.broadcast_to(b_ref[...], o_ref.shape)
    for s in range(steps):
        copy(s).wait()
        if s + AHEAD < steps:
            copy(s + AHEAD).start()
        l, kk = divmod(s, nk)
        fb = f_ref[:, kk * BLK:(kk + 1) * BLK].astype(jnp.bfloat16)
        wb = w_buf[s % NBUF].astype(jnp.bfloat16)
        o_ref[:, l * d:(l + 1) * d] += jnp.dot(
            fb, wb, preferred_element_type=jnp.float32)


def kernel(f, weight, bias):
    B, F = f.shape
    L, _, D = weight.shape
    nk = F // BLK
    w2d = weight.reshape(L * F, D)
    bias2 = bias.reshape(1, L * D)
    out = pl.pallas_call(
        functools.partial(_decode_kernel, nk=nk, d=D),
        in_specs=[
            pl.BlockSpec(memory_space=pltpu.MemorySpace.VMEM),
            pl.BlockSpec(memory_space=pltpu.MemorySpace.HBM),
            pl.BlockSpec(memory_space=pltpu.MemorySpace.VMEM),
        ],
        out_specs=pl.BlockSpec(memory_space=pltpu.MemorySpace.VMEM),
        out_shape=jax.ShapeDtypeStruct((B, L * D), jnp.float32),
        scratch_shapes=[
            pltpu.VMEM((NBUF, BLK, D), jnp.float32),
            pltpu.SemaphoreType.DMA((NBUF,)),
        ],
    )(f, w2d, bias2)
    return out.reshape(B, L, D)


# speedup vs baseline: 1.0066x; 1.0066x over previous
import functools

import jax
import jax.numpy as jnp
from jax.experimental import pallas as pl
from jax.experimental.pallas import tpu as pltpu

BLK = 1024
NBUF = 8


def _decode_kernel(f_ref, w_ref, b_ref, o_ref, w_buf, sem, *, nk: int, d: int):
    steps = 2 * nk

    def copy(s):
        return pltpu.make_async_copy(
            w_ref.at[pl.ds(s * BLK, BLK), :],
            w_buf.at[s % NBUF],
            sem.at[s % NBUF],
        )

    AHEAD = NBUF - 2
    for j in range(AHEAD):
        copy(j).start()
    o_ref[...] = jnp.broadcast_to(b_ref[...], o_ref.shape)
    for s in range(steps):
        copy(s).wait()
        if s + AHEAD < steps:
            copy(s + AHEAD).start()
        l, kk = divmod(s, nk)
        fb = f_ref[:, kk * BLK:(kk + 1) * BLK].astype(jnp.bfloat16)
        wb = w_buf[s % NBUF].astype(jnp.bfloat16)
        o_ref[:, l * d:(l + 1) * d] += jnp.dot(
            fb, wb, preferred_element_type=jnp.float32)


def kernel(f, weight, bias):
    B, F = f.shape
    L, _, D = weight.shape
    nk = F // BLK
    w2d = weight.reshape(L * F, D)
    bias2 = bias.reshape(1, L * D)
    out = pl.pallas_call(
        functools.partial(_decode_kernel, nk=nk, d=D),
        in_specs=[
            pl.BlockSpec(memory_space=pltpu.MemorySpace.VMEM),
            pl.BlockSpec(memory_space=pltpu.MemorySpace.HBM),
            pl.BlockSpec(memory_space=pltpu.MemorySpace.VMEM),
        ],
        out_specs=pl.BlockSpec(memory_space=pltpu.MemorySpace.VMEM),
        out_shape=jax.ShapeDtypeStruct((B, L * D), jnp.float32),
        scratch_shapes=[
            pltpu.VMEM((NBUF, BLK, D), jnp.float32),
            pltpu.SemaphoreType.DMA((NBUF,)),
        ],
    )(f, w2d, bias2)
    return out.reshape(B, L, D)


# speedup vs baseline: 1.0093x; 1.0027x over previous
import functools

import jax
import jax.numpy as jnp
from jax.experimental import pallas as pl
from jax.experimental.pallas import tpu as pltpu

BLK = 1024
NBUF = 8


def _decode_kernel(f_ref, w_ref, b_ref, o_ref, w_buf, sem, *,
                   nl: int, nk: int, d: int):
    steps = nl * nk

    def copy(s):
        return pltpu.make_async_copy(
            w_ref.at[pl.ds(s * BLK, BLK), :],
            w_buf.at[s % NBUF],
            sem.at[s % NBUF],
        )

    AHEAD = NBUF - 2
    for j in range(AHEAD):
        copy(j).start()
    o_ref[...] = jnp.broadcast_to(b_ref[...], o_ref.shape)
    for s in range(steps):
        copy(s).wait()
        if s + AHEAD < steps:
            copy(s + AHEAD).start()
        l, kk = divmod(s, nk)
        fb = f_ref[:, kk * BLK:(kk + 1) * BLK].astype(jnp.bfloat16)
        wb = w_buf[s % NBUF].astype(jnp.bfloat16)
        o_ref[:, l * d:(l + 1) * d] += jnp.dot(
            fb, wb, preferred_element_type=jnp.float32)


def kernel(f, weight, bias):
    B, F = f.shape
    L, _, D = weight.shape
    nk = F // BLK
    w2d = weight.reshape(L * F, D)
    bias2 = bias.reshape(1, L * D)
    out = pl.pallas_call(
        functools.partial(_decode_kernel, nl=L, nk=nk, d=D),
        in_specs=[
            pl.BlockSpec(memory_space=pltpu.MemorySpace.VMEM),
            pl.BlockSpec(memory_space=pltpu.MemorySpace.HBM),
            pl.BlockSpec(memory_space=pltpu.MemorySpace.VMEM),
        ],
        out_specs=pl.BlockSpec(memory_space=pltpu.MemorySpace.VMEM),
        out_shape=jax.ShapeDtypeStruct((B, L * D), jnp.float32),
        scratch_shapes=[
            pltpu.VMEM((NBUF, BLK, D), jnp.float32),
            pltpu.SemaphoreType.DMA((NBUF,)),
        ],
    )(f, w2d, bias2)
    return out.reshape(B, L, D)


# speedup vs baseline: 1.0119x; 1.0026x over previous
import functools

import jax
import jax.numpy as jnp
from jax.experimental import pallas as pl
from jax.experimental.pallas import tpu as pltpu

BLK = 1024
NBUF = 8


def _decode_kernel(f_ref, w_ref, b_ref, o_ref, w_buf, sem, *,
                   nl: int, nk: int, d: int):
    steps = nl * nk

    def copy(s):
        return pltpu.make_async_copy(
            w_ref.at[pl.ds(s * BLK, BLK), :],
            w_buf.at[s % NBUF],
            sem.at[s % NBUF],
        )

    AHEAD = NBUF - 4
    for j in range(AHEAD):
        copy(j).start()
    o_ref[...] = jnp.broadcast_to(b_ref[...], o_ref.shape)
    for s in range(steps):
        copy(s).wait()
        if s + AHEAD < steps:
            copy(s + AHEAD).start()
        l, kk = divmod(s, nk)
        fb = f_ref[:, kk * BLK:(kk + 1) * BLK].astype(jnp.bfloat16)
        wb = w_buf[s % NBUF].astype(jnp.bfloat16)
        o_ref[:, l * d:(l + 1) * d] += jnp.dot(
            fb, wb, preferred_element_type=jnp.float32)


def kernel(f, weight, bias):
    B, F = f.shape
    L, _, D = weight.shape
    nk = F // BLK
    w2d = weight.reshape(L * F, D)
    bias2 = bias.reshape(1, L * D)
    out = pl.pallas_call(
        functools.partial(_decode_kernel, nl=L, nk=nk, d=D),
        in_specs=[
            pl.BlockSpec(memory_space=pltpu.MemorySpace.VMEM),
            pl.BlockSpec(memory_space=pltpu.MemorySpace.HBM),
            pl.BlockSpec(memory_space=pltpu.MemorySpace.VMEM),
        ],
        out_specs=pl.BlockSpec(memory_space=pltpu.MemorySpace.VMEM),
        out_shape=jax.ShapeDtypeStruct((B, L * D), jnp.float32),
        scratch_shapes=[
            pltpu.VMEM((NBUF, BLK, D), jnp.float32),
            pltpu.SemaphoreType.DMA((NBUF,)),
        ],
    )(f, w2d, bias2)
    return out.reshape(B, L, D)
